# baseline (speedup 1.0000x reference)
import jax
import jax.numpy as jnp
from jax.experimental import pallas as pl

_N = 4096
_R = 512


def _chamfer_body(a_ref, bT_ref, fwd_ref, bwd_ref):
    i = pl.program_id(0)
    a = a_ref[...]
    bT = bT_ref[...]
    ab = a.astype(jnp.bfloat16).astype(jnp.float32)
    bTb = bT.astype(jnp.bfloat16).astype(jnp.float32)
    cross = (ab[:, 0:1] * bTb[0:1, :]
             + ab[:, 1:2] * bTb[1:2, :]
             + ab[:, 2:3] * bTb[2:3, :])
    a2 = (a[:, 0:1] * a[:, 0:1]
          + a[:, 1:2] * a[:, 1:2]
          + a[:, 2:3] * a[:, 2:3])
    b2 = (bT[0:1, :] * bT[0:1, :]
          + bT[1:2, :] * bT[1:2, :]
          + bT[2:3, :] * bT[2:3, :])
    d = jnp.maximum((a2 + b2) - 2.0 * cross, 0.0)
    fwd_ref[...] = jnp.min(d, axis=1, keepdims=True)
    colmin = jnp.min(d, axis=0, keepdims=True)

    @pl.when(i == 0)
    def _():
        bwd_ref[...] = colmin

    @pl.when(i > 0)
    def _():
        bwd_ref[...] = jnp.minimum(bwd_ref[...], colmin)


def kernel(source_cloud, target_cloud):
    src = source_cloud[0]
    tgt = target_cloud[0]
    tgtT = tgt.T

    fwd, bwd = pl.pallas_call(
        _chamfer_body,
        grid=(_N // _R,),
        in_specs=[
            pl.BlockSpec((_R, 3), lambda i: (i, 0)),
            pl.BlockSpec((3, _N), lambda i: (0, 0)),
        ],
        out_specs=[
            pl.BlockSpec((_R, 1), lambda i: (i, 0)),
            pl.BlockSpec((1, _N), lambda i: (0, 0)),
        ],
        out_shape=[
            jax.ShapeDtypeStruct((_N, 1), jnp.float32),
            jax.ShapeDtypeStruct((1, _N), jnp.float32),
        ],
    )(src, tgtT)

    return fwd.reshape(_N), bwd.reshape(_N)


# speedup vs baseline: 1.8110x; 1.8110x over previous
import jax
import jax.numpy as jnp
from jax.experimental import pallas as pl

_N = 4096
_R = 512
_K = 8


def _chamfer_body(a_ref, bT_ref, fwd_ref, bwd_ref):
    i = pl.program_id(0)
    d = jnp.dot(a_ref[...], bT_ref[...],
                preferred_element_type=jnp.float32)
    fwd_ref[...] = jnp.maximum(jnp.min(d, axis=1, keepdims=True), 0.0)
    colmin = jnp.min(d, axis=0, keepdims=True)

    last = _N // _R - 1

    @pl.when(i == 0)
    def _():
        bwd_ref[...] = colmin

    @pl.when((i > 0) & (i < last))
    def _():
        bwd_ref[...] = jnp.minimum(bwd_ref[...], colmin)

    @pl.when(i == last)
    def _():
        bwd_ref[...] = jnp.maximum(jnp.minimum(bwd_ref[...], colmin), 0.0)


def _augment(pts):
    sq = jnp.sum(pts * pts, axis=1, keepdims=True)
    sq_hi = jax.lax.bitcast_convert_type(
        jax.lax.bitcast_convert_type(sq, jnp.uint32) & jnp.uint32(0xFFFF0000),
        jnp.float32)
    sq_lo = sq - sq_hi
    ones = jnp.ones_like(sq)
    zero = jnp.zeros_like(sq)
    row = jnp.concatenate([sq_hi, sq_lo, ones, ones, -2.0 * pts, zero],
                          axis=1).astype(jnp.bfloat16)
    col = jnp.concatenate([ones, ones, sq_hi, sq_lo, pts, zero],
                          axis=1).astype(jnp.bfloat16).T
    return row, col


def kernel(source_cloud, target_cloud):
    src = source_cloud[0]
    tgt = target_cloud[0]
    a_row, _ = _augment(src)
    _, b_col = _augment(tgt)

    fwd, bwd = pl.pallas_call(
        _chamfer_body,
        grid=(_N // _R,),
        in_specs=[
            pl.BlockSpec((_R, _K), lambda i: (i, 0)),
            pl.BlockSpec((_K, _N), lambda i: (0, 0)),
        ],
        out_specs=[
            pl.BlockSpec((_R, 1), lambda i: (i, 0)),
            pl.BlockSpec((1, _N), lambda i: (0, 0)),
        ],
        out_shape=[
            jax.ShapeDtypeStruct((_N, 1), jnp.float32),
            jax.ShapeDtypeStruct((1, _N), jnp.float32),
        ],
    )(a_row, b_col)

    return fwd.reshape(_N), bwd.reshape(_N)
